# baseline (speedup 1.0000x reference)
import functools

import jax
import jax.numpy as jnp
from jax import lax
from jax.experimental import pallas as pl
from jax.experimental.pallas import tpu as pltpu
from jax.experimental.pallas import tpu_sc as plsc

_LANES = 16
_ROW = 128
_NW = 32
_ROWS_PER_W = 16
_FRAME_ROWS = _NW * _ROWS_PER_W


def _sc_gather(table, row_idx):
    mesh = plsc.VectorSubcoreMesh(core_axis_name="c", subcore_axis_name="s")

    @functools.partial(
        pl.kernel,
        mesh=mesh,
        out_type=jax.ShapeDtypeStruct((_FRAME_ROWS, _ROW), jnp.float32),
        scratch_types=[
            pltpu.VMEM((_LANES,), jnp.int32),
            pltpu.VMEM((_ROWS_PER_W, _ROW), jnp.float32),
            pltpu.SemaphoreType.DMA,
        ],
    )
    def k(table_hbm, idx_hbm, out_hbm, idx_v, rows_v, sem):
        wid = lax.axis_index("s") * 2 + lax.axis_index("c")
        pltpu.sync_copy(idx_hbm.at[wid], idx_v)
        pltpu.async_copy(table_hbm.at[idx_v], rows_v, sem).wait()
        pltpu.sync_copy(rows_v, out_hbm.at[pl.ds(wid * _ROWS_PER_W, _ROWS_PER_W)])

    return k(table, row_idx)


def kernel(noise, idx):
    length = noise.shape[0]
    table = noise.reshape(length * _FRAME_ROWS, _ROW)
    base = (jnp.asarray(idx, jnp.int32) % length) * _FRAME_ROWS
    row_idx = (base + jnp.arange(_FRAME_ROWS, dtype=jnp.int32)).reshape(
        _NW, _ROWS_PER_W
    )
    out = _sc_gather(table, row_idx)
    return out.reshape(1, *noise.shape[1:])
